# Initial kernel scaffold; baseline (speedup 1.0000x reference)
import functools

import jax
import jax.numpy as jnp
from jax import lax
from jax.experimental import pallas as pl
from jax.experimental.pallas import tpu as pltpu
from jax.experimental.pallas import tpu_sc as plsc

NC = 2
NS = 16
NW = NC * NS

CHUNK = 128


def _gather_body(n_chunks, idx_hbm, table_hbm, out_hbm, idx_v, buf, sem):
    wid = lax.axis_index("s") * NC + lax.axis_index("c")
    row0 = wid * n_chunks
    pltpu.sync_copy(idx_hbm.at[pl.ds(row0, n_chunks)], idx_v)

    def chunk(c, _):
        pltpu.async_copy(table_hbm.at[idx_v.at[c]], buf, sem).wait()
        off = pl.multiple_of((row0 + c) * CHUNK, CHUNK)
        pltpu.sync_copy(buf, out_hbm.at[pl.ds(off, CHUNK)])
        return 0

    lax.fori_loop(0, n_chunks, chunk, 0)


def kernel(indices, table):
    batch, hist = indices.shape
    vocab, embed = table.shape
    total = batch * hist
    assert total % (NW * CHUNK) == 0
    n_chunks = total // (NW * CHUNK)

    idx2d = indices.reshape(total // CHUNK, CHUNK).astype(jnp.int32)

    mesh = plsc.VectorSubcoreMesh(core_axis_name="c", subcore_axis_name="s")
    run = pl.kernel(
        functools.partial(_gather_body, n_chunks),
        out_type=jax.ShapeDtypeStruct((total, embed), jnp.float32),
        mesh=mesh,
        scratch_types=[
            pltpu.VMEM((n_chunks, CHUNK), jnp.int32),
            pltpu.VMEM((CHUNK, embed), jnp.float32),
            pltpu.SemaphoreType.DMA,
        ],
    )
    out = run(idx2d, table)
    return out.reshape(batch, hist, embed)


# speedup vs baseline: 2.9737x; 2.9737x over previous
import functools

import jax
import jax.numpy as jnp
from jax import lax
from jax.experimental import pallas as pl
from jax.experimental.pallas import tpu as pltpu
from jax.experimental.pallas import tpu_sc as plsc

NC = 2
NS = 16
NW = NC * NS

CHUNK = 128


def _gather_body(n_chunks, idx_hbm, table_hbm, out_hbm, idx_v, buf, sem):
    wid = lax.axis_index("s") * NC + lax.axis_index("c")
    per_w = n_chunks * CHUNK
    base = wid * per_w
    pltpu.sync_copy(idx_hbm.at[pl.ds(base, per_w)], idx_v)

    def chunk(c, _):
        idx_slice = idx_v.at[pl.ds(pl.multiple_of(c * CHUNK, CHUNK), CHUNK)]
        pltpu.async_copy(table_hbm.at[idx_slice], buf, sem).wait()
        off = pl.multiple_of(base + c * CHUNK, CHUNK)
        pltpu.sync_copy(buf, out_hbm.at[pl.ds(off, CHUNK)])
        return 0

    lax.fori_loop(0, n_chunks, chunk, 0)


def kernel(indices, table):
    batch, hist = indices.shape
    vocab, embed = table.shape
    total = batch * hist
    assert total % (NW * CHUNK) == 0
    n_chunks = total // (NW * CHUNK)

    idx_flat = indices.reshape(total).astype(jnp.int32)

    mesh = plsc.VectorSubcoreMesh(core_axis_name="c", subcore_axis_name="s")
    run = pl.kernel(
        functools.partial(_gather_body, n_chunks),
        out_type=jax.ShapeDtypeStruct((total, embed), jnp.float32),
        mesh=mesh,
        scratch_types=[
            pltpu.VMEM((n_chunks * CHUNK,), jnp.int32),
            pltpu.VMEM((CHUNK, embed), jnp.float32),
            pltpu.SemaphoreType.DMA,
        ],
    )
    out = run(idx_flat, table)
    return out.reshape(batch, hist, embed)


# speedup vs baseline: 3.3409x; 1.1235x over previous
import functools

import jax
import jax.numpy as jnp
from jax import lax
from jax.experimental import pallas as pl
from jax.experimental.pallas import tpu as pltpu
from jax.experimental.pallas import tpu_sc as plsc

NC = 2
NS = 16
NW = NC * NS

CHUNK = 128


NBUF = 5


def _gather_body(n_chunks, idx_hbm, table_hbm, out_hbm, idx_v, bufs, gsem, wsem):
    wid = lax.axis_index("s") * NC + lax.axis_index("c")
    per_w = n_chunks * CHUNK
    base = wid * per_w
    pltpu.sync_copy(idx_hbm.at[pl.ds(base, per_w)], idx_v)

    def gather_cp(c, j):
        idx_slice = idx_v.at[pl.ds(pl.multiple_of(c * CHUNK, CHUNK), CHUNK)]
        return pltpu.make_async_copy(table_hbm.at[idx_slice], bufs.at[j], gsem.at[j])

    def wb_cp(c, j):
        off = pl.multiple_of(base + c * CHUNK, CHUNK)
        return pltpu.make_async_copy(bufs.at[j], out_hbm.at[pl.ds(off, CHUNK)], wsem.at[j])

    gather_cp(0, 0).start()

    def outer(i, _):
        g = i * NBUF
        for j in range(NBUF):
            c = g + j
            nxt = c + 1
            bn = (j + 1) % NBUF

            @pl.when((nxt >= NBUF) & (nxt < n_chunks))
            def _():
                wb_cp(nxt - NBUF, bn).wait()

            @pl.when(nxt < n_chunks)
            def _():
                gather_cp(nxt, bn).start()

            gather_cp(c, j).wait()
            wb_cp(c, j).start()
        return 0

    lax.fori_loop(0, n_chunks // NBUF, outer, 0)

    for c in range(n_chunks - NBUF, n_chunks):
        wb_cp(c, c % NBUF).wait()


def kernel(indices, table):
    batch, hist = indices.shape
    vocab, embed = table.shape
    total = batch * hist
    assert total % (NW * CHUNK) == 0
    n_chunks = total // (NW * CHUNK)

    idx_flat = indices.reshape(total).astype(jnp.int32)

    mesh = plsc.VectorSubcoreMesh(core_axis_name="c", subcore_axis_name="s")
    run = pl.kernel(
        functools.partial(_gather_body, n_chunks),
        out_type=jax.ShapeDtypeStruct((total, embed), jnp.float32),
        mesh=mesh,
        scratch_types=[
            pltpu.VMEM((n_chunks * CHUNK,), jnp.int32),
            pltpu.VMEM((NBUF, CHUNK, embed), jnp.float32),
            pltpu.SemaphoreType.DMA((NBUF,)),
            pltpu.SemaphoreType.DMA((NBUF,)),
        ],
    )
    out = run(idx_flat, table)
    return out.reshape(batch, hist, embed)
